# baseline (speedup 1.0000x reference)
import functools

import jax
import jax.numpy as jnp
import numpy as np
from jax import lax
from jax.experimental import pallas as pl
from jax.experimental.pallas import tpu as pltpu
from jax.experimental.pallas import tpu_sc as plsc

_N_IMAGES = 16
_OBJ = 256
_N_TOTAL = _N_IMAGES * _OBJ
_EMBED = 256
_N_EMB = 600
_G = min(max(40000 // _N_TOTAL // 2, 1), 10)
_PATTERN_ROWS = _OBJ * 2
_REPLICAS = _N_IMAGES * _G
_OUT_ROWS = _REPLICAS * _PATTERN_ROWS

_NC, _NS = 2, 16
_NW = _NC * _NS
_K_REPL = 16
_NGRP = _REPLICAS // _K_REPL
_W = _NW // _NGRP
_R = _PATTERN_ROWS // _W


def _threefry2x32(k0, k1, x0, x1):
    def rotl(v, d):
        return (v << np.uint32(d)) | (v >> np.uint32(32 - d))
    ks = [k0, k1, np.uint32(k0 ^ k1 ^ np.uint32(0x1BD11BDA))]
    x0 = (x0 + k0).astype(np.uint32)
    x1 = (x1 + k1).astype(np.uint32)
    rotations = ((13, 15, 26, 6), (17, 29, 16, 24))
    for i in range(5):
        for d in rotations[i % 2]:
            x0 = (x0 + x1).astype(np.uint32)
            x1 = rotl(x1, d)
            x1 = x1 ^ x0
        x0 = (x0 + ks[(i + 1) % 3]).astype(np.uint32)
        x1 = (x1 + ks[(i + 2) % 3] + np.uint32(i + 1)).astype(np.uint32)
    return x0, x1


def _random_bits(key, size):
    o0, o1 = _threefry2x32(key[0], key[1], np.zeros(size, np.uint32),
                           np.arange(size, dtype=np.uint32))
    return o0 ^ o1


def _split(key, num):
    o0, o1 = _threefry2x32(key[0], key[1], np.zeros(num, np.uint32),
                           np.arange(num, dtype=np.uint32))
    return np.stack([o0, o1], axis=1)


def _uniform(key, size, lo, hi):
    bits = _random_bits(key, size)
    f = ((bits >> np.uint32(9)) | np.uint32(0x3F800000)).view(np.float32)
    f = (f - np.float32(1.0)).astype(np.float32)
    lo32, hi32 = np.float32(lo), np.float32(hi)
    return np.maximum(lo32, f * (hi32 - lo32) + lo32).astype(np.float32)


def _erfinv32(x):
    w = (-np.log1p((-x * x).astype(np.float32))).astype(np.float32)
    w_s = (w - np.float32(2.5)).astype(np.float32)
    p_s = np.full_like(x, 2.81022636e-08)
    for c in (3.43273939e-07, -3.5233877e-06, -4.39150654e-06, 0.00021858087,
              -0.00125372503, -0.00417768164, 0.246640727, 1.50140941):
        p_s = (p_s * w_s + np.float32(c)).astype(np.float32)
    w_l = (np.sqrt(np.maximum(w, np.float32(0))) - np.float32(3.0)).astype(np.float32)
    p_l = np.full_like(x, -0.000200214257)
    for c in (0.000100950558, 0.00134934322, -0.00367342844, 0.00573950773,
              -0.0076224613, 0.00943887047, 1.00167406, 2.83297682):
        p_l = (p_l * w_l + np.float32(c)).astype(np.float32)
    p = np.where(w < np.float32(5.0), p_s, p_l)
    return (p * x).astype(np.float32)


def _normal(key, size):
    lo = np.nextafter(np.float32(-1.0), np.float32(0.0), dtype=np.float32)
    u = _uniform(key, size, lo, np.float32(1.0))
    return (np.float32(np.sqrt(2)) * _erfinv32(u)).astype(np.float32)


def _permutation(key, n):
    x = np.arange(n)
    num_rounds = int(np.ceil(3 * np.log(max(1, n)) / np.log(2**32 - 1)))
    for _ in range(num_rounds):
        key, subkey = _split(key, 2)
        sort_keys = _random_bits(subkey, n)
        x = x[np.argsort(sort_keys, kind="stable")]
    return x


def _fold_constants():
    key = _split(np.array([0, 42], dtype=np.uint32), 3)
    kn, ku, kp = key[0], key[1], key[2]
    noise = _normal(kn, _G * _N_TOTAL * 4).reshape(_G, _N_TOTAL, 2, 2)
    noise = (noise * np.float32(2.0)).astype(np.float32)
    norm = np.sqrt((noise * noise).sum(axis=-1, keepdims=True).astype(np.float32)
                   ).astype(np.float32)
    pos_norm = norm[:, :, 0]
    neg_norm = norm[:, :, 1]
    neg_scale = _uniform(ku, _G * _N_TOTAL, 1.0, 2.0).reshape(_G, _N_TOTAL, 1)
    neg_scale = ((neg_scale / (neg_norm + np.float32(1e-06))) * pos_norm
                 ).astype(np.float32)
    noise[:, :, 1] = noise[:, :, 1] * neg_scale
    noise_t = noise.reshape(_G, _N_IMAGES, _OBJ, 2, 2).transpose(1, 0, 2, 3, 4)
    perm = _permutation(kp, _N_EMB)[:_PATTERN_ROWS]
    return (np.ascontiguousarray(noise_t, dtype=np.float32).reshape(_N_IMAGES, _G, _OBJ, 4),
            np.ascontiguousarray(perm).astype(np.int32))


_NOISE_NP, _PERM_NP = _fold_constants()


def _sc_queries_body(table_hbm, idx_hbm, out_hbm, idx_v, rows_v, gsem, wsem):
    wid = lax.axis_index("s") * _NC + lax.axis_index("c")
    gi = wid // _W
    li = wid % _W
    base = li * _R
    pltpu.sync_copy(idx_hbm.at[pl.ds(base, _R)], idx_v)
    pltpu.async_copy(table_hbm.at[idx_v], rows_v, gsem).wait()
    copies = []
    for k in range(_K_REPL):
        off = (gi * _K_REPL + k) * _PATTERN_ROWS + base
        copies.append(pltpu.async_copy(rows_v, out_hbm.at[pl.ds(off, _R)], wsem))
    for c in copies:
        c.wait()


@functools.cache
def _sc_queries():
    return pl.kernel(
        _sc_queries_body,
        out_type=jax.ShapeDtypeStruct((_OUT_ROWS, _EMBED), jnp.float32),
        mesh=plsc.VectorSubcoreMesh(core_axis_name="c", subcore_axis_name="s",
                                    num_cores=_NC, num_subcores=_NS),
        scratch_types=[
            pltpu.VMEM((_R,), jnp.int32),
            pltpu.VMEM((_R, _EMBED), jnp.float32),
            pltpu.SemaphoreType.DMA,
            pltpu.SemaphoreType.DMA,
        ],
    )


def _tc_noised_body(pos_ref, size_ref, noise_ref, out_ref):
    pos = pos_ref[...]
    pc = jnp.concatenate([pos, pos], axis=-1)
    x = pc[None, None] + noise_ref[...]
    sz = size_ref[...].reshape(1, 1, 1, 2)
    mx = jnp.concatenate([sz, sz], axis=-1)
    out_ref[...] = jnp.clip(x, 0.0, mx)


_tc_noised = pl.pallas_call(
    _tc_noised_body,
    grid=(_N_IMAGES,),
    in_specs=[
        pl.BlockSpec((_OBJ, 2), lambda i: (i, 0)),
        pl.BlockSpec((1, 1, 2), lambda i: (i, 0, 0)),
        pl.BlockSpec((1, _G, _OBJ, 4), lambda i: (i, 0, 0, 0)),
    ],
    out_specs=pl.BlockSpec((1, _G, _OBJ, 4), lambda i: (i, 0, 0, 0)),
    out_shape=jax.ShapeDtypeStruct((_N_IMAGES, _G, _OBJ, 4), jnp.float32),
)


def kernel(incidence_points_pixels_rc, image_size_pixels_rc, batch_size,
           electron_batch_offsets, dn_embedding_weight):
    del batch_size, electron_batch_offsets
    out_q = _sc_queries()(dn_embedding_weight, jnp.asarray(_PERM_NP))
    out_n = _tc_noised(incidence_points_pixels_rc,
                       image_size_pixels_rc.reshape(_N_IMAGES, 1, 2),
                       jnp.asarray(_NOISE_NP))
    return (out_q.reshape(_N_IMAGES, _G, _OBJ, 2, _EMBED),
            out_n.reshape(_N_IMAGES, _G, _OBJ, 2, 2))


# speedup vs baseline: 1.3050x; 1.3050x over previous
import functools

import jax
import jax.numpy as jnp
import numpy as np
from jax import lax
from jax.experimental import pallas as pl
from jax.experimental.pallas import tpu as pltpu
from jax.experimental.pallas import tpu_sc as plsc

_N_IMAGES = 16
_OBJ = 256
_N_TOTAL = _N_IMAGES * _OBJ
_EMBED = 256
_N_EMB = 600
_G = min(max(40000 // _N_TOTAL // 2, 1), 10)
_PATTERN_ROWS = _OBJ * 2
_REPLICAS = _N_IMAGES * _G
_OUT_ROWS = _REPLICAS * _PATTERN_ROWS

_NC, _NS = 2, 16
_NW = _NC * _NS
_K_REPL = 16
_NGRP = _REPLICAS // _K_REPL
_W = _NW // _NGRP
_R = _PATTERN_ROWS // _W


def _threefry2x32(k0, k1, x0, x1):
    def rotl(v, d):
        return (v << np.uint32(d)) | (v >> np.uint32(32 - d))
    ks = [k0, k1, np.uint32(k0 ^ k1 ^ np.uint32(0x1BD11BDA))]
    x0 = (x0 + k0).astype(np.uint32)
    x1 = (x1 + k1).astype(np.uint32)
    rotations = ((13, 15, 26, 6), (17, 29, 16, 24))
    for i in range(5):
        for d in rotations[i % 2]:
            x0 = (x0 + x1).astype(np.uint32)
            x1 = rotl(x1, d)
            x1 = x1 ^ x0
        x0 = (x0 + ks[(i + 1) % 3]).astype(np.uint32)
        x1 = (x1 + ks[(i + 2) % 3] + np.uint32(i + 1)).astype(np.uint32)
    return x0, x1


def _random_bits(key, size):
    o0, o1 = _threefry2x32(key[0], key[1], np.zeros(size, np.uint32),
                           np.arange(size, dtype=np.uint32))
    return o0 ^ o1


def _split(key, num):
    o0, o1 = _threefry2x32(key[0], key[1], np.zeros(num, np.uint32),
                           np.arange(num, dtype=np.uint32))
    return np.stack([o0, o1], axis=1)


def _uniform(key, size, lo, hi):
    bits = _random_bits(key, size)
    f = ((bits >> np.uint32(9)) | np.uint32(0x3F800000)).view(np.float32)
    f = (f - np.float32(1.0)).astype(np.float32)
    lo32, hi32 = np.float32(lo), np.float32(hi)
    return np.maximum(lo32, f * (hi32 - lo32) + lo32).astype(np.float32)


def _erfinv32(x):
    w = (-np.log1p((-x * x).astype(np.float32))).astype(np.float32)
    w_s = (w - np.float32(2.5)).astype(np.float32)
    p_s = np.full_like(x, 2.81022636e-08)
    for c in (3.43273939e-07, -3.5233877e-06, -4.39150654e-06, 0.00021858087,
              -0.00125372503, -0.00417768164, 0.246640727, 1.50140941):
        p_s = (p_s * w_s + np.float32(c)).astype(np.float32)
    w_l = (np.sqrt(np.maximum(w, np.float32(0))) - np.float32(3.0)).astype(np.float32)
    p_l = np.full_like(x, -0.000200214257)
    for c in (0.000100950558, 0.00134934322, -0.00367342844, 0.00573950773,
              -0.0076224613, 0.00943887047, 1.00167406, 2.83297682):
        p_l = (p_l * w_l + np.float32(c)).astype(np.float32)
    p = np.where(w < np.float32(5.0), p_s, p_l)
    return (p * x).astype(np.float32)


def _normal(key, size):
    lo = np.nextafter(np.float32(-1.0), np.float32(0.0), dtype=np.float32)
    u = _uniform(key, size, lo, np.float32(1.0))
    return (np.float32(np.sqrt(2)) * _erfinv32(u)).astype(np.float32)


def _permutation(key, n):
    x = np.arange(n)
    num_rounds = int(np.ceil(3 * np.log(max(1, n)) / np.log(2**32 - 1)))
    for _ in range(num_rounds):
        key, subkey = _split(key, 2)
        sort_keys = _random_bits(subkey, n)
        x = x[np.argsort(sort_keys, kind="stable")]
    return x


def _fold_constants():
    key = _split(np.array([0, 42], dtype=np.uint32), 3)
    kn, ku, kp = key[0], key[1], key[2]
    noise = _normal(kn, _G * _N_TOTAL * 4).reshape(_G, _N_TOTAL, 2, 2)
    noise = (noise * np.float32(2.0)).astype(np.float32)
    norm = np.sqrt((noise * noise).sum(axis=-1, keepdims=True).astype(np.float32)
                   ).astype(np.float32)
    pos_norm = norm[:, :, 0]
    neg_norm = norm[:, :, 1]
    neg_scale = _uniform(ku, _G * _N_TOTAL, 1.0, 2.0).reshape(_G, _N_TOTAL, 1)
    neg_scale = ((neg_scale / (neg_norm + np.float32(1e-06))) * pos_norm
                 ).astype(np.float32)
    noise[:, :, 1] = noise[:, :, 1] * neg_scale
    noise_t = noise.reshape(_G, _N_IMAGES, _OBJ, 2, 2).transpose(1, 0, 2, 3, 4)
    perm = _permutation(kp, _N_EMB)[:_PATTERN_ROWS]
    return (np.ascontiguousarray(noise_t, dtype=np.float32).reshape(_N_IMAGES, _G, _OBJ, 4),
            np.ascontiguousarray(perm).astype(np.int32))


_NOISE_NP, _PERM_NP = _fold_constants()


_OC = _R // 2


def _sc_queries_body(table_hbm, idx0_hbm, idx1_hbm, out_hbm,
                     idx0_v, idx1_v, rows0_v, rows1_v, gsem, wsem):
    wid = lax.axis_index("s") * _NC + lax.axis_index("c")
    gi = wid // _W
    li = wid % _W
    obase = li * _OC
    pltpu.sync_copy(idx0_hbm.at[pl.ds(obase, _OC)], idx0_v)
    pltpu.sync_copy(idx1_hbm.at[pl.ds(obase, _OC)], idx1_v)
    g0 = pltpu.async_copy(table_hbm.at[idx0_v], rows0_v, gsem)
    g1 = pltpu.async_copy(table_hbm.at[idx1_v], rows1_v, gsem)
    g0.wait()
    g1.wait()
    copies = []
    for k in range(_K_REPL):
        r = gi * _K_REPL + k
        img, g = r // _G, r % _G
        copies.append(pltpu.async_copy(
            rows0_v, out_hbm.at[img, g, pl.ds(obase, _OC), 0, :], wsem))
        copies.append(pltpu.async_copy(
            rows1_v, out_hbm.at[img, g, pl.ds(obase, _OC), 1, :], wsem))
    for c in copies:
        c.wait()


@functools.cache
def _sc_queries():
    return pl.kernel(
        _sc_queries_body,
        out_type=jax.ShapeDtypeStruct((_N_IMAGES, _G, _OBJ, 2, _EMBED),
                                      jnp.float32),
        mesh=plsc.VectorSubcoreMesh(core_axis_name="c", subcore_axis_name="s",
                                    num_cores=_NC, num_subcores=_NS),
        scratch_types=[
            pltpu.VMEM((_OC,), jnp.int32),
            pltpu.VMEM((_OC,), jnp.int32),
            pltpu.VMEM((_OC, _EMBED), jnp.float32),
            pltpu.VMEM((_OC, _EMBED), jnp.float32),
            pltpu.SemaphoreType.DMA,
            pltpu.SemaphoreType.DMA,
        ],
    )


def _tc_noised_body(pos_ref, size_ref, noise_ref, out_ref):
    pos = pos_ref[...]
    pc = jnp.concatenate([pos, pos], axis=-1)
    x = pc[None] + noise_ref[0]
    sz = size_ref[...].reshape(1, 1, 2)
    mx = jnp.concatenate([sz, sz], axis=-1)
    y = jnp.clip(x, 0.0, mx)
    out_ref[0] = y.reshape(_G, _OBJ, 2, 2)


_tc_noised = pl.pallas_call(
    _tc_noised_body,
    grid=(_N_IMAGES,),
    in_specs=[
        pl.BlockSpec((_OBJ, 2), lambda i: (i, 0)),
        pl.BlockSpec((1, 1, 2), lambda i: (i, 0, 0)),
        pl.BlockSpec((1, _G, _OBJ, 4), lambda i: (i, 0, 0, 0)),
    ],
    out_specs=pl.BlockSpec((1, _G, _OBJ, 2, 2), lambda i: (i, 0, 0, 0, 0)),
    out_shape=jax.ShapeDtypeStruct((_N_IMAGES, _G, _OBJ, 2, 2), jnp.float32),
)


def kernel(incidence_points_pixels_rc, image_size_pixels_rc, batch_size,
           electron_batch_offsets, dn_embedding_weight):
    del batch_size, electron_batch_offsets
    out_q = _sc_queries()(dn_embedding_weight,
                          jnp.asarray(_PERM_NP[0::2]),
                          jnp.asarray(_PERM_NP[1::2]))
    out_n = _tc_noised(incidence_points_pixels_rc,
                       image_size_pixels_rc.reshape(_N_IMAGES, 1, 2),
                       jnp.asarray(_NOISE_NP))
    return (out_q, out_n)


# speedup vs baseline: 2.5321x; 1.9403x over previous
import functools

import jax
import jax.numpy as jnp
import numpy as np
from jax import lax
from jax.experimental import pallas as pl
from jax.experimental.pallas import tpu as pltpu
from jax.experimental.pallas import tpu_sc as plsc

_N_IMAGES = 16
_OBJ = 256
_N_TOTAL = _N_IMAGES * _OBJ
_EMBED = 256
_N_EMB = 600
_G = min(max(40000 // _N_TOTAL // 2, 1), 10)
_PATTERN_ROWS = _OBJ * 2
_REPLICAS = _N_IMAGES * _G
_OUT_ROWS = _REPLICAS * _PATTERN_ROWS

_NC, _NS = 2, 16
_NW = _NC * _NS
_K_REPL = 16
_NGRP = _REPLICAS // _K_REPL
_W = _NW // _NGRP
_R = _PATTERN_ROWS // _W


def _threefry2x32(k0, k1, x0, x1):
    def rotl(v, d):
        return (v << np.uint32(d)) | (v >> np.uint32(32 - d))
    ks = [k0, k1, np.uint32(k0 ^ k1 ^ np.uint32(0x1BD11BDA))]
    x0 = (x0 + k0).astype(np.uint32)
    x1 = (x1 + k1).astype(np.uint32)
    rotations = ((13, 15, 26, 6), (17, 29, 16, 24))
    for i in range(5):
        for d in rotations[i % 2]:
            x0 = (x0 + x1).astype(np.uint32)
            x1 = rotl(x1, d)
            x1 = x1 ^ x0
        x0 = (x0 + ks[(i + 1) % 3]).astype(np.uint32)
        x1 = (x1 + ks[(i + 2) % 3] + np.uint32(i + 1)).astype(np.uint32)
    return x0, x1


def _random_bits(key, size):
    o0, o1 = _threefry2x32(key[0], key[1], np.zeros(size, np.uint32),
                           np.arange(size, dtype=np.uint32))
    return o0 ^ o1


def _split(key, num):
    o0, o1 = _threefry2x32(key[0], key[1], np.zeros(num, np.uint32),
                           np.arange(num, dtype=np.uint32))
    return np.stack([o0, o1], axis=1)


def _uniform(key, size, lo, hi):
    bits = _random_bits(key, size)
    f = ((bits >> np.uint32(9)) | np.uint32(0x3F800000)).view(np.float32)
    f = (f - np.float32(1.0)).astype(np.float32)
    lo32, hi32 = np.float32(lo), np.float32(hi)
    return np.maximum(lo32, f * (hi32 - lo32) + lo32).astype(np.float32)


def _erfinv32(x):
    w = (-np.log1p((-x * x).astype(np.float32))).astype(np.float32)
    w_s = (w - np.float32(2.5)).astype(np.float32)
    p_s = np.full_like(x, 2.81022636e-08)
    for c in (3.43273939e-07, -3.5233877e-06, -4.39150654e-06, 0.00021858087,
              -0.00125372503, -0.00417768164, 0.246640727, 1.50140941):
        p_s = (p_s * w_s + np.float32(c)).astype(np.float32)
    w_l = (np.sqrt(np.maximum(w, np.float32(0))) - np.float32(3.0)).astype(np.float32)
    p_l = np.full_like(x, -0.000200214257)
    for c in (0.000100950558, 0.00134934322, -0.00367342844, 0.00573950773,
              -0.0076224613, 0.00943887047, 1.00167406, 2.83297682):
        p_l = (p_l * w_l + np.float32(c)).astype(np.float32)
    p = np.where(w < np.float32(5.0), p_s, p_l)
    return (p * x).astype(np.float32)


def _normal(key, size):
    lo = np.nextafter(np.float32(-1.0), np.float32(0.0), dtype=np.float32)
    u = _uniform(key, size, lo, np.float32(1.0))
    return (np.float32(np.sqrt(2)) * _erfinv32(u)).astype(np.float32)


def _permutation(key, n):
    x = np.arange(n)
    num_rounds = int(np.ceil(3 * np.log(max(1, n)) / np.log(2**32 - 1)))
    for _ in range(num_rounds):
        key, subkey = _split(key, 2)
        sort_keys = _random_bits(subkey, n)
        x = x[np.argsort(sort_keys, kind="stable")]
    return x


def _fold_constants():
    key = _split(np.array([0, 42], dtype=np.uint32), 3)
    kn, ku, kp = key[0], key[1], key[2]
    noise = _normal(kn, _G * _N_TOTAL * 4).reshape(_G, _N_TOTAL, 2, 2)
    noise = (noise * np.float32(2.0)).astype(np.float32)
    norm = np.sqrt((noise * noise).sum(axis=-1, keepdims=True).astype(np.float32)
                   ).astype(np.float32)
    pos_norm = norm[:, :, 0]
    neg_norm = norm[:, :, 1]
    neg_scale = _uniform(ku, _G * _N_TOTAL, 1.0, 2.0).reshape(_G, _N_TOTAL, 1)
    neg_scale = ((neg_scale / (neg_norm + np.float32(1e-06))) * pos_norm
                 ).astype(np.float32)
    noise[:, :, 1] = noise[:, :, 1] * neg_scale
    noise_t = noise.reshape(_G, _N_IMAGES, _OBJ, 2, 2).transpose(1, 0, 3, 4, 2)
    perm = _permutation(kp, _N_EMB)[:_PATTERN_ROWS]
    return (np.ascontiguousarray(noise_t, dtype=np.float32),
            np.ascontiguousarray(perm).astype(np.int32))


_NOISE_NP, _PERM_NP = _fold_constants()


_OC = _R // 2


def _sc_queries_body(table_hbm, idx0_hbm, idx1_hbm, out_hbm,
                     idx0_v, idx1_v, rows0_v, rows1_v, gsem, wsem):
    wid = lax.axis_index("s") * _NC + lax.axis_index("c")
    gi = wid // _W
    li = wid % _W
    obase = li * _OC
    pltpu.sync_copy(idx0_hbm.at[pl.ds(obase, _OC)], idx0_v)
    pltpu.sync_copy(idx1_hbm.at[pl.ds(obase, _OC)], idx1_v)
    g0 = pltpu.async_copy(table_hbm.at[idx0_v], rows0_v, gsem)
    g1 = pltpu.async_copy(table_hbm.at[idx1_v], rows1_v, gsem)
    g0.wait()
    g1.wait()
    copies = []
    for k in range(_K_REPL):
        r = gi * _K_REPL + k
        img, g = r // _G, r % _G
        copies.append(pltpu.async_copy(
            rows0_v, out_hbm.at[img, g, pl.ds(obase, _OC), 0, :], wsem))
        copies.append(pltpu.async_copy(
            rows1_v, out_hbm.at[img, g, pl.ds(obase, _OC), 1, :], wsem))
    for c in copies:
        c.wait()


@functools.cache
def _sc_queries():
    return pl.kernel(
        _sc_queries_body,
        out_type=jax.ShapeDtypeStruct((_N_IMAGES, _G, _OBJ, 2, _EMBED),
                                      jnp.float32),
        mesh=plsc.VectorSubcoreMesh(core_axis_name="c", subcore_axis_name="s",
                                    num_cores=_NC, num_subcores=_NS),
        scratch_types=[
            pltpu.VMEM((_OC,), jnp.int32),
            pltpu.VMEM((_OC,), jnp.int32),
            pltpu.VMEM((_OC, _EMBED), jnp.float32),
            pltpu.VMEM((_OC, _EMBED), jnp.float32),
            pltpu.SemaphoreType.DMA,
            pltpu.SemaphoreType.DMA,
        ],
    )


def _tc_noised_body(pos_ref, size_ref, noise_ref, out_ref):
    pos = pos_ref[0]
    x = pos[None, None] + noise_ref[0]
    mx = size_ref[...].reshape(1, 1, 2, 1)
    out_ref[0] = jnp.clip(x, 0.0, mx)


_tc_noised = pl.pallas_call(
    _tc_noised_body,
    grid=(_N_IMAGES,),
    in_specs=[
        pl.BlockSpec((1, 2, _OBJ), lambda i: (i, 0, 0)),
        pl.BlockSpec((1, 1, 2), lambda i: (i, 0, 0)),
        pl.BlockSpec((1, _G, 2, 2, _OBJ), lambda i: (i, 0, 0, 0, 0)),
    ],
    out_specs=pl.BlockSpec((1, _G, 2, 2, _OBJ), lambda i: (i, 0, 0, 0, 0)),
    out_shape=jax.ShapeDtypeStruct((_N_IMAGES, _G, 2, 2, _OBJ), jnp.float32),
)


def kernel(incidence_points_pixels_rc, image_size_pixels_rc, batch_size,
           electron_batch_offsets, dn_embedding_weight):
    del batch_size, electron_batch_offsets
    out_q = _sc_queries()(dn_embedding_weight,
                          jnp.asarray(_PERM_NP[0::2]),
                          jnp.asarray(_PERM_NP[1::2]))
    pos_t = jnp.transpose(incidence_points_pixels_rc.reshape(_N_IMAGES, _OBJ, 2),
                          (0, 2, 1))
    out_n5 = _tc_noised(pos_t,
                        image_size_pixels_rc.reshape(_N_IMAGES, 1, 2),
                        jnp.asarray(_NOISE_NP))
    out_n = jnp.transpose(out_n5, (0, 1, 4, 2, 3))
    return (out_q, out_n)


# speedup vs baseline: 2.6100x; 1.0308x over previous
import functools

import jax
import jax.numpy as jnp
import numpy as np
from jax import lax
from jax.experimental import pallas as pl
from jax.experimental.pallas import tpu as pltpu
from jax.experimental.pallas import tpu_sc as plsc

_N_IMAGES = 16
_OBJ = 256
_N_TOTAL = _N_IMAGES * _OBJ
_EMBED = 256
_N_EMB = 600
_G = min(max(40000 // _N_TOTAL // 2, 1), 10)
_PATTERN_ROWS = _OBJ * 2
_REPLICAS = _N_IMAGES * _G
_OUT_ROWS = _REPLICAS * _PATTERN_ROWS

_NC, _NS = 2, 16
_NW = _NC * _NS
_K_REPL = 16
_NGRP = _REPLICAS // _K_REPL
_W = _NW // _NGRP
_R = _PATTERN_ROWS // _W


def _threefry2x32(k0, k1, x0, x1):
    def rotl(v, d):
        return (v << np.uint32(d)) | (v >> np.uint32(32 - d))
    ks = [k0, k1, np.uint32(k0 ^ k1 ^ np.uint32(0x1BD11BDA))]
    x0 = (x0 + k0).astype(np.uint32)
    x1 = (x1 + k1).astype(np.uint32)
    rotations = ((13, 15, 26, 6), (17, 29, 16, 24))
    for i in range(5):
        for d in rotations[i % 2]:
            x0 = (x0 + x1).astype(np.uint32)
            x1 = rotl(x1, d)
            x1 = x1 ^ x0
        x0 = (x0 + ks[(i + 1) % 3]).astype(np.uint32)
        x1 = (x1 + ks[(i + 2) % 3] + np.uint32(i + 1)).astype(np.uint32)
    return x0, x1


def _random_bits(key, size):
    o0, o1 = _threefry2x32(key[0], key[1], np.zeros(size, np.uint32),
                           np.arange(size, dtype=np.uint32))
    return o0 ^ o1


def _split(key, num):
    o0, o1 = _threefry2x32(key[0], key[1], np.zeros(num, np.uint32),
                           np.arange(num, dtype=np.uint32))
    return np.stack([o0, o1], axis=1)


def _uniform(key, size, lo, hi):
    bits = _random_bits(key, size)
    f = ((bits >> np.uint32(9)) | np.uint32(0x3F800000)).view(np.float32)
    f = (f - np.float32(1.0)).astype(np.float32)
    lo32, hi32 = np.float32(lo), np.float32(hi)
    return np.maximum(lo32, f * (hi32 - lo32) + lo32).astype(np.float32)


def _erfinv32(x):
    w = (-np.log1p((-x * x).astype(np.float32))).astype(np.float32)
    w_s = (w - np.float32(2.5)).astype(np.float32)
    p_s = np.full_like(x, 2.81022636e-08)
    for c in (3.43273939e-07, -3.5233877e-06, -4.39150654e-06, 0.00021858087,
              -0.00125372503, -0.00417768164, 0.246640727, 1.50140941):
        p_s = (p_s * w_s + np.float32(c)).astype(np.float32)
    w_l = (np.sqrt(np.maximum(w, np.float32(0))) - np.float32(3.0)).astype(np.float32)
    p_l = np.full_like(x, -0.000200214257)
    for c in (0.000100950558, 0.00134934322, -0.00367342844, 0.00573950773,
              -0.0076224613, 0.00943887047, 1.00167406, 2.83297682):
        p_l = (p_l * w_l + np.float32(c)).astype(np.float32)
    p = np.where(w < np.float32(5.0), p_s, p_l)
    return (p * x).astype(np.float32)


def _normal(key, size):
    lo = np.nextafter(np.float32(-1.0), np.float32(0.0), dtype=np.float32)
    u = _uniform(key, size, lo, np.float32(1.0))
    return (np.float32(np.sqrt(2)) * _erfinv32(u)).astype(np.float32)


def _permutation(key, n):
    x = np.arange(n)
    num_rounds = int(np.ceil(3 * np.log(max(1, n)) / np.log(2**32 - 1)))
    for _ in range(num_rounds):
        key, subkey = _split(key, 2)
        sort_keys = _random_bits(subkey, n)
        x = x[np.argsort(sort_keys, kind="stable")]
    return x


def _fold_constants():
    key = _split(np.array([0, 42], dtype=np.uint32), 3)
    kn, ku, kp = key[0], key[1], key[2]
    noise = _normal(kn, _G * _N_TOTAL * 4).reshape(_G, _N_TOTAL, 2, 2)
    noise = (noise * np.float32(2.0)).astype(np.float32)
    norm = np.sqrt((noise * noise).sum(axis=-1, keepdims=True).astype(np.float32)
                   ).astype(np.float32)
    pos_norm = norm[:, :, 0]
    neg_norm = norm[:, :, 1]
    neg_scale = _uniform(ku, _G * _N_TOTAL, 1.0, 2.0).reshape(_G, _N_TOTAL, 1)
    neg_scale = ((neg_scale / (neg_norm + np.float32(1e-06))) * pos_norm
                 ).astype(np.float32)
    noise[:, :, 1] = noise[:, :, 1] * neg_scale
    noise_t = noise.reshape(_G, _N_IMAGES, _OBJ, 2, 2).transpose(1, 0, 3, 4, 2)
    perm = _permutation(kp, _N_EMB)[:_PATTERN_ROWS]
    return (np.ascontiguousarray(noise_t, dtype=np.float32),
            np.ascontiguousarray(perm).astype(np.int32))


_NOISE_NP, _PERM_NP = _fold_constants()


_OC = _R // 2


def _sc_queries_body(table_hbm, idx0_hbm, idx1_hbm, out_hbm,
                     idx0_v, idx1_v, rows0_v, rows1_v, gsem, wsem):
    wid = lax.axis_index("s") * _NC + lax.axis_index("c")
    gi = wid // _W
    li = wid % _W
    obase = li * _OC
    pltpu.sync_copy(idx0_hbm.at[pl.ds(obase, _OC)], idx0_v)
    pltpu.sync_copy(idx1_hbm.at[pl.ds(obase, _OC)], idx1_v)
    g0 = pltpu.async_copy(table_hbm.at[idx0_v], rows0_v, gsem)
    g1 = pltpu.async_copy(table_hbm.at[idx1_v], rows1_v, gsem)
    g0.wait()
    g1.wait()

    def _issue(k, carry):
        r = gi * _K_REPL + k
        img, g = r // _G, r % _G
        pltpu.async_copy(rows0_v, out_hbm.at[img, g, pl.ds(obase, _OC), 0, :],
                         wsem)
        pltpu.async_copy(rows1_v, out_hbm.at[img, g, pl.ds(obase, _OC), 1, :],
                         wsem)
        return carry

    lax.fori_loop(0, _K_REPL, _issue, 0)

    def _drain(k, carry):
        pltpu.make_async_copy(
            rows0_v, out_hbm.at[0, 0, pl.ds(obase, _OC), 0, :], wsem).wait()
        return carry

    lax.fori_loop(0, 2 * _K_REPL, _drain, 0)


@functools.cache
def _sc_queries():
    return pl.kernel(
        _sc_queries_body,
        out_type=jax.ShapeDtypeStruct((_N_IMAGES, _G, _OBJ, 2, _EMBED),
                                      jnp.float32),
        mesh=plsc.VectorSubcoreMesh(core_axis_name="c", subcore_axis_name="s",
                                    num_cores=_NC, num_subcores=_NS),
        scratch_types=[
            pltpu.VMEM((_OC,), jnp.int32),
            pltpu.VMEM((_OC,), jnp.int32),
            pltpu.VMEM((_OC, _EMBED), jnp.float32),
            pltpu.VMEM((_OC, _EMBED), jnp.float32),
            pltpu.SemaphoreType.DMA,
            pltpu.SemaphoreType.DMA,
        ],
    )


def _tc_noised_body(pos_ref, size_ref, noise_ref, out_ref):
    pos = pos_ref[...]
    x = pos[:, None, None] + noise_ref[...]
    mx = size_ref[...].reshape(_N_IMAGES, 1, 1, 2, 1)
    out_ref[...] = jnp.clip(x, 0.0, mx)


_tc_noised = pl.pallas_call(
    _tc_noised_body,
    out_shape=jax.ShapeDtypeStruct((_N_IMAGES, _G, 2, 2, _OBJ), jnp.float32),
)


def kernel(incidence_points_pixels_rc, image_size_pixels_rc, batch_size,
           electron_batch_offsets, dn_embedding_weight):
    del batch_size, electron_batch_offsets
    out_q = _sc_queries()(dn_embedding_weight,
                          jnp.asarray(_PERM_NP[0::2]),
                          jnp.asarray(_PERM_NP[1::2]))
    pos_t = jnp.transpose(incidence_points_pixels_rc.reshape(_N_IMAGES, _OBJ, 2),
                          (0, 2, 1))
    out_n5 = _tc_noised(pos_t,
                        image_size_pixels_rc.reshape(_N_IMAGES, 1, 2),
                        jnp.asarray(_NOISE_NP))
    out_n = jnp.transpose(out_n5, (0, 1, 4, 2, 3))
    return (out_q, out_n)


# speedup vs baseline: 2.6495x; 1.0151x over previous
import functools

import jax
import jax.numpy as jnp
import numpy as np
from jax import lax
from jax.experimental import pallas as pl
from jax.experimental.pallas import tpu as pltpu
from jax.experimental.pallas import tpu_sc as plsc

_N_IMAGES = 16
_OBJ = 256
_N_TOTAL = _N_IMAGES * _OBJ
_EMBED = 256
_N_EMB = 600
_G = min(max(40000 // _N_TOTAL // 2, 1), 10)
_PATTERN_ROWS = _OBJ * 2
_REPLICAS = _N_IMAGES * _G
_OUT_ROWS = _REPLICAS * _PATTERN_ROWS

_NC, _NS = 2, 16
_NW = _NC * _NS
_K_REPL = 16
_NGRP = _REPLICAS // _K_REPL
_W = _NW // _NGRP
_R = _PATTERN_ROWS // _W


def _threefry2x32(k0, k1, x0, x1):
    def rotl(v, d):
        return (v << np.uint32(d)) | (v >> np.uint32(32 - d))
    ks = [k0, k1, np.uint32(k0 ^ k1 ^ np.uint32(0x1BD11BDA))]
    x0 = (x0 + k0).astype(np.uint32)
    x1 = (x1 + k1).astype(np.uint32)
    rotations = ((13, 15, 26, 6), (17, 29, 16, 24))
    for i in range(5):
        for d in rotations[i % 2]:
            x0 = (x0 + x1).astype(np.uint32)
            x1 = rotl(x1, d)
            x1 = x1 ^ x0
        x0 = (x0 + ks[(i + 1) % 3]).astype(np.uint32)
        x1 = (x1 + ks[(i + 2) % 3] + np.uint32(i + 1)).astype(np.uint32)
    return x0, x1


def _random_bits(key, size):
    o0, o1 = _threefry2x32(key[0], key[1], np.zeros(size, np.uint32),
                           np.arange(size, dtype=np.uint32))
    return o0 ^ o1


def _split(key, num):
    o0, o1 = _threefry2x32(key[0], key[1], np.zeros(num, np.uint32),
                           np.arange(num, dtype=np.uint32))
    return np.stack([o0, o1], axis=1)


def _uniform(key, size, lo, hi):
    bits = _random_bits(key, size)
    f = ((bits >> np.uint32(9)) | np.uint32(0x3F800000)).view(np.float32)
    f = (f - np.float32(1.0)).astype(np.float32)
    lo32, hi32 = np.float32(lo), np.float32(hi)
    return np.maximum(lo32, f * (hi32 - lo32) + lo32).astype(np.float32)


def _erfinv32(x):
    w = (-np.log1p((-x * x).astype(np.float32))).astype(np.float32)
    w_s = (w - np.float32(2.5)).astype(np.float32)
    p_s = np.full_like(x, 2.81022636e-08)
    for c in (3.43273939e-07, -3.5233877e-06, -4.39150654e-06, 0.00021858087,
              -0.00125372503, -0.00417768164, 0.246640727, 1.50140941):
        p_s = (p_s * w_s + np.float32(c)).astype(np.float32)
    w_l = (np.sqrt(np.maximum(w, np.float32(0))) - np.float32(3.0)).astype(np.float32)
    p_l = np.full_like(x, -0.000200214257)
    for c in (0.000100950558, 0.00134934322, -0.00367342844, 0.00573950773,
              -0.0076224613, 0.00943887047, 1.00167406, 2.83297682):
        p_l = (p_l * w_l + np.float32(c)).astype(np.float32)
    p = np.where(w < np.float32(5.0), p_s, p_l)
    return (p * x).astype(np.float32)


def _normal(key, size):
    lo = np.nextafter(np.float32(-1.0), np.float32(0.0), dtype=np.float32)
    u = _uniform(key, size, lo, np.float32(1.0))
    return (np.float32(np.sqrt(2)) * _erfinv32(u)).astype(np.float32)


def _permutation(key, n):
    x = np.arange(n)
    num_rounds = int(np.ceil(3 * np.log(max(1, n)) / np.log(2**32 - 1)))
    for _ in range(num_rounds):
        key, subkey = _split(key, 2)
        sort_keys = _random_bits(subkey, n)
        x = x[np.argsort(sort_keys, kind="stable")]
    return x


def _fold_constants():
    key = _split(np.array([0, 42], dtype=np.uint32), 3)
    kn, ku, kp = key[0], key[1], key[2]
    noise = _normal(kn, _G * _N_TOTAL * 4).reshape(_G, _N_TOTAL, 2, 2)
    noise = (noise * np.float32(2.0)).astype(np.float32)
    norm = np.sqrt((noise * noise).sum(axis=-1, keepdims=True).astype(np.float32)
                   ).astype(np.float32)
    pos_norm = norm[:, :, 0]
    neg_norm = norm[:, :, 1]
    neg_scale = _uniform(ku, _G * _N_TOTAL, 1.0, 2.0).reshape(_G, _N_TOTAL, 1)
    neg_scale = ((neg_scale / (neg_norm + np.float32(1e-06))) * pos_norm
                 ).astype(np.float32)
    noise[:, :, 1] = noise[:, :, 1] * neg_scale
    noise_t = noise.reshape(_G, _N_IMAGES, _OBJ, 2, 2).transpose(1, 0, 3, 4, 2)
    perm = _permutation(kp, _N_EMB)[:_PATTERN_ROWS]
    return (np.ascontiguousarray(noise_t, dtype=np.float32),
            np.ascontiguousarray(perm).astype(np.int32))


_NOISE_NP, _PERM_NP = _fold_constants()


_OC = _R // 2


def _sc_queries_body(table_hbm, idx_hbm, out_hbm,
                     idx0_v, idx1_v, rows0_v, rows1_v, gsem, wsem):
    wid = lax.axis_index("s") * _NC + lax.axis_index("c")
    gi = wid // _W
    li = wid % _W
    obase = li * _OC
    pltpu.sync_copy(idx_hbm.at[0, pl.ds(obase, _OC)], idx0_v)
    pltpu.sync_copy(idx_hbm.at[1, pl.ds(obase, _OC)], idx1_v)
    g0 = pltpu.async_copy(table_hbm.at[idx0_v], rows0_v, gsem)
    g1 = pltpu.async_copy(table_hbm.at[idx1_v], rows1_v, gsem)
    g0.wait()
    g1.wait()

    def _issue(k, carry):
        r = gi * _K_REPL + k
        img, g = r // _G, r % _G
        pltpu.async_copy(rows0_v, out_hbm.at[img, g, pl.ds(obase, _OC), 0, :],
                         wsem)
        pltpu.async_copy(rows1_v, out_hbm.at[img, g, pl.ds(obase, _OC), 1, :],
                         wsem)
        return carry

    lax.fori_loop(0, _K_REPL, _issue, 0)

    def _drain(k, carry):
        pltpu.make_async_copy(
            rows0_v, out_hbm.at[0, 0, pl.ds(obase, _OC), 0, :], wsem).wait()
        return carry

    lax.fori_loop(0, 2 * _K_REPL, _drain, 0)


@functools.cache
def _sc_queries():
    return pl.kernel(
        _sc_queries_body,
        out_type=jax.ShapeDtypeStruct((_N_IMAGES, _G, _OBJ, 2, _EMBED),
                                      jnp.float32),
        mesh=plsc.VectorSubcoreMesh(core_axis_name="c", subcore_axis_name="s",
                                    num_cores=_NC, num_subcores=_NS),
        scratch_types=[
            pltpu.VMEM((_OC,), jnp.int32),
            pltpu.VMEM((_OC,), jnp.int32),
            pltpu.VMEM((_OC, _EMBED), jnp.float32),
            pltpu.VMEM((_OC, _EMBED), jnp.float32),
            pltpu.SemaphoreType.DMA,
            pltpu.SemaphoreType.DMA,
        ],
    )


def _tc_noised_body(pos_ref, size_ref, noise_ref, out_ref):
    pos = pos_ref[...]
    x = pos[:, None, None] + noise_ref[...]
    mx = size_ref[...].reshape(_N_IMAGES, 1, 1, 2, 1)
    out_ref[...] = jnp.clip(x, 0.0, mx)


_tc_noised = pl.pallas_call(
    _tc_noised_body,
    out_shape=jax.ShapeDtypeStruct((_N_IMAGES, _G, 2, 2, _OBJ), jnp.float32),
)


def kernel(incidence_points_pixels_rc, image_size_pixels_rc, batch_size,
           electron_batch_offsets, dn_embedding_weight):
    del batch_size, electron_batch_offsets
    out_q = _sc_queries()(dn_embedding_weight,
                          jnp.asarray(np.stack([_PERM_NP[0::2],
                                                _PERM_NP[1::2]])))
    pos_t = jnp.transpose(incidence_points_pixels_rc.reshape(_N_IMAGES, _OBJ, 2),
                          (0, 2, 1))
    out_n5 = _tc_noised(pos_t,
                        image_size_pixels_rc.reshape(_N_IMAGES, 1, 2),
                        jnp.asarray(_NOISE_NP))
    out_n = jnp.transpose(out_n5, (0, 1, 4, 2, 3))
    return (out_q, out_n)
